# baseline (speedup 1.0000x reference)
import numpy as np
import jax
import jax.numpy as jnp
from jax import lax
from jax.experimental import pallas as pl
from jax.experimental.pallas import tpu as pltpu

_B, _S, _D, _M, _K = 8, 16, 64, 16384, 8
_CM = 512
_NC = _M // _CM
_NEG = np.float32(-1e30)


def _mem_layer_body(xs_ref, memT_ref, wqT_ref, wk_ref, wvT_ref,
                    wu1T_ref, wu2T_ref, bq_ref, bk_ref, bv_ref, bu_ref,
                    out_ref, cur_ref, lg_ref, cm_ref):
    def init_chunk(c, carry):
        sl = pl.ds(c * _CM, _CM)
        cur_ref[:, :, sl] = jnp.broadcast_to(
            memT_ref[:, sl][None], (_B, _D, _CM))
        return carry
    lax.fori_loop(0, _NC, init_chunk, 0)

    scale = np.float32(np.sqrt(_D))
    iota_d = lax.broadcasted_iota(jnp.int32, (_D, _D), 0)
    ident = (iota_d == lax.broadcasted_iota(jnp.int32, (_D, _D), 1)
             ).astype(jnp.float32)
    iotc = lax.broadcasted_iota(jnp.int32, (1, _NC), 1)
    iot_cm = lax.broadcasted_iota(jnp.int32, (1, _CM), 1)
    iot128 = lax.broadcasted_iota(jnp.int32, (1, 128), 1)

    def step(t, carry0):
        xt = xs_ref[t]
        q = jnp.dot(xt, wqT_ref[...],
                    preferred_element_type=jnp.float32) + bq_ref[...]
        qk = jnp.dot(q, wk_ref[...],
                     preferred_element_type=jnp.float32)
        qbk = jnp.sum(q * bk_ref[...], axis=1, keepdims=True)

        mxs = []
        for b in range(_B):
            l_b = (jnp.dot(qk[b:b + 1, :], cur_ref[b],
                           preferred_element_type=jnp.float32)
                   + qbk[b:b + 1, :]) / scale
            lg_ref[b:b + 1, :] = l_b
            mxs.append(jnp.max(l_b, axis=1, keepdims=True))
        mx = jnp.concatenate(mxs, axis=0)

        def p2(c, carry):
            ssum = carry
            sl = pl.ds(c * _CM, _CM)
            e_c = jnp.exp(lg_ref[:, sl] - mx)
            lg_ref[:, sl] = e_c
            cmax = jnp.max(e_c, axis=1, keepdims=True)
            cm_ref[...] = jnp.where(iotc == c, cmax, cm_ref[...])
            return ssum + jnp.sum(e_c, axis=1, keepdims=True)
        ssum = lax.fori_loop(0, _NC, p2, jnp.zeros((_B, 1), jnp.float32))

        s = jnp.concatenate(
            [lax.dot_general(lg_ref[b:b + 1, :], cur_ref[b],
                             (((1,), (1,)), ((), ())),
                             preferred_element_type=jnp.float32)
             for b in range(_B)], axis=0)

        mem_out = jnp.dot(s / ssum, wvT_ref[...],
                          preferred_element_type=jnp.float32) + bv_ref[...]
        out_ref[t] = mem_out

        us = jax.nn.sigmoid(
            jnp.dot(xt, wu1T_ref[...], preferred_element_type=jnp.float32)
            + jnp.dot(mem_out, wu2T_ref[...], preferred_element_type=jnp.float32)
            + bu_ref[...])
        usT = lax.dot_general(ident, us, (((1,), (1,)), ((), ())),
                              preferred_element_type=jnp.float32)
        xtT = lax.dot_general(ident, xt, (((1,), (1,)), ((), ())),
                              preferred_element_type=jnp.float32)

        def tk(i, carry1):
            for b in range(_B):
                row = cm_ref[b:b + 1, :]
                cmax = jnp.max(row, axis=1, keepdims=True)
                cs = jnp.min(jnp.where(row == cmax, iotc, _NC),
                             axis=1, keepdims=True)[0, 0]
                base = cs * _CM
                chunk = lg_ref[b:b + 1, pl.ds(base, _CM)]
                lmax = jnp.max(chunk, axis=1, keepdims=True)
                li = jnp.min(jnp.where(chunk == lmax, iot_cm, _CM),
                             axis=1, keepdims=True)[0, 0]
                gidx = base + li
                sub = lax.div(gidx, 128)
                loff = lax.rem(gidx, 128)
                cur_sub = cur_ref[b, :, pl.ds(sub * 128, 128)]
                selm = (iot128 == loff).astype(jnp.float32)
                gate = usT[:, b:b + 1] * selm
                cur_ref[b, :, pl.ds(sub * 128, 128)] = (
                    cur_sub + gate * (xtT[:, b:b + 1] - cur_sub))
                chunk2 = jnp.where(iot_cm == li, _NEG, chunk)
                lg_ref[b:b + 1, pl.ds(base, _CM)] = chunk2
                newcm = jnp.max(chunk2, axis=1, keepdims=True)
                cm_ref[b:b + 1, :] = jnp.where(iotc == cs, newcm, row)
            return carry1
        lax.fori_loop(0, _K, tk, 0)
        return carry0

    lax.fori_loop(0, _S, step, 0)


def kernel(x, memory, Wq, bq, Wk, bk, Wv, bv, Wu, bu):
    xs = jnp.transpose(x, (1, 0, 2))
    memT = jnp.transpose(memory)
    outs = pl.pallas_call(
        _mem_layer_body,
        out_shape=jax.ShapeDtypeStruct((_S, _B, _D), jnp.float32),
        scratch_shapes=[pltpu.VMEM((_B, _D, _M), jnp.float32),
                        pltpu.VMEM((_B, _M), jnp.float32),
                        pltpu.VMEM((_B, _NC), jnp.float32)],
        compiler_params=pltpu.CompilerParams(
            vmem_limit_bytes=62 * 1024 * 1024),
    )(xs, memT, Wq.T, Wk, Wv.T, Wu[:, :_D].T, Wu[:, _D:].T,
      bq.reshape(1, _D), bk.reshape(1, _D), bv.reshape(1, _D),
      bu.reshape(1, _D))
    return jnp.transpose(outs, (1, 0, 2))


# speedup vs baseline: 1.0091x; 1.0091x over previous
import numpy as np
import jax
import jax.numpy as jnp
from jax import lax
from jax.experimental import pallas as pl
from jax.experimental.pallas import tpu as pltpu

_B, _S, _D, _M, _K = 8, 16, 64, 16384, 8
_CM = 512
_NC = _M // _CM
_CW = 4096
_NW = _M // _CW
_NEG = np.float32(-1e30)


def _mem_layer_body(xs_ref, memT_ref, wqT_ref, wk_ref, wvT_ref,
                    wu1T_ref, wu2T_ref, bq_ref, bk_ref, bv_ref, bu_ref,
                    out_ref, cur_ref, lg_ref, mk_ref):
    def init_chunk(c, carry):
        sl = pl.ds(c * _CM, _CM)
        cur_ref[:, :, sl] = jnp.broadcast_to(
            memT_ref[:, sl][None], (_B, _D, _CM))
        mk_ref[:, sl] = jnp.zeros((_B, _CM), jnp.float32)
        return carry
    lax.fori_loop(0, _NC, init_chunk, 0)

    scale = np.float32(np.sqrt(_D))

    def step(t, carry0):
        us_p, xt_p = carry0
        xt = xs_ref[t]
        q = jnp.dot(xt, wqT_ref[...],
                    preferred_element_type=jnp.float32) + bq_ref[...]
        qk = jnp.dot(q, wk_ref[...],
                     preferred_element_type=jnp.float32)
        qbk = jnp.sum(q * bk_ref[...], axis=1, keepdims=True)

        def p1(c, mx):
            sl = pl.ds(c * _CM, _CM)
            cur_c = cur_ref[:, :, sl]
            gate = us_p[:, :, None] * mk_ref[:, sl][:, None, :]
            cur_n = cur_c + gate * (xt_p[:, :, None] - cur_c)
            cur_ref[:, :, sl] = cur_n
            mk_ref[:, sl] = jnp.zeros((_B, _CM), jnp.float32)
            mxs = []
            for b in range(_B):
                l_b = (jnp.dot(qk[b:b + 1, :], cur_n[b],
                               preferred_element_type=jnp.float32)
                       + qbk[b:b + 1, :]) / scale
                lg_ref[b:b + 1, sl] = l_b
                mxs.append(jnp.max(l_b, axis=1, keepdims=True))
            return jnp.maximum(mx, jnp.concatenate(mxs, axis=0))
        mx = lax.fori_loop(0, _NC, p1,
                           jnp.full((_B, 1), _NEG, jnp.float32))

        def p2(c, carry):
            ssum, s = carry
            sl = pl.ds(c * _CM, _CM)
            e_c = jnp.exp(lg_ref[:, sl] - mx)
            lg_ref[:, sl] = e_c
            cur_c = cur_ref[:, :, sl]
            return (ssum + jnp.sum(e_c, axis=1, keepdims=True),
                    s + jnp.sum(cur_c * e_c[:, None, :], axis=2))
        ssum, s = lax.fori_loop(
            0, _NC, p2,
            (jnp.zeros((_B, 1), jnp.float32),
             jnp.zeros((_B, _D), jnp.float32)))

        mem_out = jnp.dot(s / ssum, wvT_ref[...],
                          preferred_element_type=jnp.float32) + bv_ref[...]
        out_ref[t] = mem_out

        us = jax.nn.sigmoid(
            jnp.dot(xt, wu1T_ref[...], preferred_element_type=jnp.float32)
            + jnp.dot(mem_out, wu2T_ref[...], preferred_element_type=jnp.float32)
            + bu_ref[...])

        def tk_iter(i, carry1):
            def scan_c(c, carry):
                bestv, besti = carry
                sl = pl.ds(c * _CW, _CW)
                w = jnp.where(mk_ref[:, sl] > 0, _NEG, lg_ref[:, sl])
                iot = (lax.broadcasted_iota(jnp.int32, (_B, _CW), 1)
                       + c * _CW)
                cm = jnp.max(w, axis=1, keepdims=True)
                ci = jnp.min(jnp.where(w == cm, iot, _M), axis=1,
                             keepdims=True)
                tie = cm == bestv
                besti = jnp.where(cm > bestv, ci,
                                  jnp.where(tie, jnp.minimum(besti, ci),
                                            besti))
                bestv = jnp.maximum(bestv, cm)
                return bestv, besti
            _, besti = lax.fori_loop(
                0, _NW, scan_c,
                (jnp.full((_B, 1), _NEG, jnp.float32),
                 jnp.full((_B, 1), _M, jnp.int32)))

            def mark_c(c, carry):
                sl = pl.ds(c * _CW, _CW)
                iot = (lax.broadcasted_iota(jnp.int32, (_B, _CW), 1)
                       + c * _CW)
                mk_ref[:, sl] = jnp.where(iot == besti, 1.0, mk_ref[:, sl])
                return carry
            lax.fori_loop(0, _NW, mark_c, 0)
            return carry1
        lax.fori_loop(0, _K, tk_iter, 0)
        return us, xt

    lax.fori_loop(0, _S, step,
                  (jnp.zeros((_B, _D), jnp.float32),
                   jnp.zeros((_B, _D), jnp.float32)))


def kernel(x, memory, Wq, bq, Wk, bk, Wv, bv, Wu, bu):
    xs = jnp.transpose(x, (1, 0, 2))
    memT = jnp.transpose(memory)
    outs = pl.pallas_call(
        _mem_layer_body,
        out_shape=jax.ShapeDtypeStruct((_S, _B, _D), jnp.float32),
        scratch_shapes=[pltpu.VMEM((_B, _D, _M), jnp.float32),
                        pltpu.VMEM((_B, _M), jnp.float32),
                        pltpu.VMEM((_B, _M), jnp.float32)],
        compiler_params=pltpu.CompilerParams(
            vmem_limit_bytes=62 * 1024 * 1024),
    )(xs, memT, Wq.T, Wk, Wv.T, Wu[:, :_D].T, Wu[:, _D:].T,
      bq.reshape(1, _D), bk.reshape(1, _D), bv.reshape(1, _D),
      bu.reshape(1, _D))
    return jnp.transpose(outs, (1, 0, 2))


# speedup vs baseline: 1.5900x; 1.5755x over previous
import numpy as np
import jax
import jax.numpy as jnp
from jax import lax
from jax.experimental import pallas as pl
from jax.experimental.pallas import tpu as pltpu

_B, _S, _D, _M, _K = 8, 16, 64, 16384, 8
_CM = 1024
_NC = _M // _CM
_CW = 8192
_NW = _M // _CW
_NEG = np.float32(-1e30)


def _mem_layer_body(xs_ref, memT_ref, wqT_ref, wk_ref, wvT_ref,
                    wu1T_ref, wu2T_ref, bq_ref, bk_ref, bv_ref, bu_ref,
                    out_ref, cur_ref, lg_ref, mk_ref):
    def init_chunk(c, carry):
        sl = pl.ds(c * _CM, _CM)
        cur_ref[:, :, sl] = jnp.broadcast_to(
            memT_ref[:, sl][None], (_B, _D, _CM))
        return carry
    lax.fori_loop(0, _NC, init_chunk, 0)

    scale = np.float32(np.sqrt(_D))

    def step(t, carry0):
        xt = xs_ref[t]
        q = jnp.dot(xt, wqT_ref[...],
                    preferred_element_type=jnp.float32) + bq_ref[...]
        qk = jnp.dot(q, wk_ref[...],
                     preferred_element_type=jnp.float32)
        qbk = jnp.sum(q * bk_ref[...], axis=1, keepdims=True)

        mxs = []
        for b in range(_B):
            l_b = (jnp.dot(qk[b:b + 1, :], cur_ref[b],
                           preferred_element_type=jnp.float32)
                   + qbk[b:b + 1, :]) / scale
            lg_ref[b:b + 1, :] = l_b
            mxs.append(jnp.max(l_b, axis=1, keepdims=True))
        mx = jnp.concatenate(mxs, axis=0)

        def pz(c, carry):
            sl = pl.ds(c * _CM, _CM)
            mk_ref[:, sl] = jnp.zeros((_B, _CM), jnp.float32)
            return carry
        lax.fori_loop(0, _NC, pz, 0)

        def p2(c, carry):
            ssum, s = carry
            sl = pl.ds(c * _CM, _CM)
            e_c = jnp.exp(lg_ref[:, sl] - mx)
            lg_ref[:, sl] = e_c
            cur_c = cur_ref[:, :, sl]
            return (ssum + jnp.sum(e_c, axis=1, keepdims=True),
                    s + jnp.sum(cur_c * e_c[:, None, :], axis=2))
        ssum, s = lax.fori_loop(
            0, _NC, p2,
            (jnp.zeros((_B, 1), jnp.float32),
             jnp.zeros((_B, _D), jnp.float32)))

        mem_out = jnp.dot(s / ssum, wvT_ref[...],
                          preferred_element_type=jnp.float32) + bv_ref[...]
        out_ref[t] = mem_out

        us = jax.nn.sigmoid(
            jnp.dot(xt, wu1T_ref[...], preferred_element_type=jnp.float32)
            + jnp.dot(mem_out, wu2T_ref[...], preferred_element_type=jnp.float32)
            + bu_ref[...])

        def tk_iter(i, carry1):
            def scan_c(c, carry):
                bestv, besti = carry
                sl = pl.ds(c * _CW, _CW)
                w = jnp.where(mk_ref[:, sl] > 0, _NEG, lg_ref[:, sl])
                iot = (lax.broadcasted_iota(jnp.int32, (_B, _CW), 1)
                       + c * _CW)
                cm = jnp.max(w, axis=1, keepdims=True)
                ci = jnp.min(jnp.where(w == cm, iot, _M), axis=1,
                             keepdims=True)
                tie = cm == bestv
                besti = jnp.where(cm > bestv, ci,
                                  jnp.where(tie, jnp.minimum(besti, ci),
                                            besti))
                bestv = jnp.maximum(bestv, cm)
                return bestv, besti
            _, besti = lax.fori_loop(
                0, _NW, scan_c,
                (jnp.full((_B, 1), _NEG, jnp.float32),
                 jnp.full((_B, 1), _M, jnp.int32)))

            def mark_c(c, carry):
                sl = pl.ds(c * _CW, _CW)
                iot = (lax.broadcasted_iota(jnp.int32, (_B, _CW), 1)
                       + c * _CW)
                mk_ref[:, sl] = jnp.where(iot == besti, 1.0, mk_ref[:, sl])
                return carry
            lax.fori_loop(0, _NW, mark_c, 0)
            return carry1
        lax.fori_loop(0, _K, tk_iter, 0)

        def p4(c, carry):
            sl = pl.ds(c * _CM, _CM)
            cur_c = cur_ref[:, :, sl]
            gate = us[:, :, None] * mk_ref[:, sl][:, None, :]
            cur_ref[:, :, sl] = cur_c + gate * (xt[:, :, None] - cur_c)
            return carry
        lax.fori_loop(0, _NC, p4, 0)
        return carry0

    lax.fori_loop(0, _S, step, 0)


def kernel(x, memory, Wq, bq, Wk, bk, Wv, bv, Wu, bu):
    xs = jnp.transpose(x, (1, 0, 2))
    memT = jnp.transpose(memory)
    outs = pl.pallas_call(
        _mem_layer_body,
        out_shape=jax.ShapeDtypeStruct((_S, _B, _D), jnp.float32),
        scratch_shapes=[pltpu.VMEM((_B, _D, _M), jnp.float32),
                        pltpu.VMEM((_B, _M), jnp.float32),
                        pltpu.VMEM((_B, _M), jnp.float32)],
        compiler_params=pltpu.CompilerParams(
            vmem_limit_bytes=62 * 1024 * 1024),
    )(xs, memT, Wq.T, Wk, Wv.T, Wu[:, :_D].T, Wu[:, _D:].T,
      bq.reshape(1, _D), bk.reshape(1, _D), bv.reshape(1, _D),
      bu.reshape(1, _D))
    return jnp.transpose(outs, (1, 0, 2))


# speedup vs baseline: 1.8425x; 1.1588x over previous
import numpy as np
import jax
import jax.numpy as jnp
from jax import lax
from jax.experimental import pallas as pl
from jax.experimental.pallas import tpu as pltpu

_B, _S, _D, _M, _K = 8, 16, 64, 16384, 8
_CM = 2048
_NC = _M // _CM
_CW = 16384
_NW = _M // _CW
_NEG = np.float32(-1e30)


def _mem_layer_body(xs_ref, memT_ref, wqT_ref, wk_ref, wvT_ref,
                    wu1T_ref, wu2T_ref, bq_ref, bk_ref, bv_ref, bu_ref,
                    out_ref, cur_ref, lg_ref, mk_ref):
    def init_chunk(c, carry):
        sl = pl.ds(c * _CM, _CM)
        cur_ref[:, :, sl] = jnp.broadcast_to(
            memT_ref[:, sl][None], (_B, _D, _CM))
        return carry
    lax.fori_loop(0, _NC, init_chunk, 0)

    scale = np.float32(np.sqrt(_D))

    def step(t, carry0):
        xt = xs_ref[t]
        q = jnp.dot(xt, wqT_ref[...],
                    preferred_element_type=jnp.float32) + bq_ref[...]
        qk = jnp.dot(q, wk_ref[...],
                     preferred_element_type=jnp.float32)
        qbk = jnp.sum(q * bk_ref[...], axis=1, keepdims=True)

        mxs = []
        for b in range(_B):
            l_b = (jnp.dot(qk[b:b + 1, :], cur_ref[b],
                           preferred_element_type=jnp.float32)
                   + qbk[b:b + 1, :]) / scale
            lg_ref[b:b + 1, :] = l_b
            mxs.append(jnp.max(l_b, axis=1, keepdims=True))
        mx = jnp.concatenate(mxs, axis=0)

        def pz(c, carry):
            sl = pl.ds(c * _CM, _CM)
            mk_ref[:, sl] = jnp.zeros((_B, _CM), jnp.float32)
            return carry
        lax.fori_loop(0, _NC, pz, 0)

        def p2(c, carry):
            ssum, s = carry
            sl = pl.ds(c * _CM, _CM)
            e_c = jnp.exp(lg_ref[:, sl] - mx)
            lg_ref[:, sl] = e_c
            cur_c = cur_ref[:, :, sl]
            return (ssum + jnp.sum(e_c, axis=1, keepdims=True),
                    s + jnp.sum(cur_c * e_c[:, None, :], axis=2))
        ssum, s = lax.fori_loop(
            0, _NC, p2,
            (jnp.zeros((_B, 1), jnp.float32),
             jnp.zeros((_B, _D), jnp.float32)))

        mem_out = jnp.dot(s / ssum, wvT_ref[...],
                          preferred_element_type=jnp.float32) + bv_ref[...]
        out_ref[t] = mem_out

        us = jax.nn.sigmoid(
            jnp.dot(xt, wu1T_ref[...], preferred_element_type=jnp.float32)
            + jnp.dot(mem_out, wu2T_ref[...], preferred_element_type=jnp.float32)
            + bu_ref[...])

        def tk_iter(i, carry1):
            def scan_c(c, carry):
                bestv, besti = carry
                sl = pl.ds(c * _CW, _CW)
                w = jnp.where(mk_ref[:, sl] > 0, _NEG, lg_ref[:, sl])
                iot = (lax.broadcasted_iota(jnp.int32, (_B, _CW), 1)
                       + c * _CW)
                cm = jnp.max(w, axis=1, keepdims=True)
                ci = jnp.min(jnp.where(w == cm, iot, _M), axis=1,
                             keepdims=True)
                tie = cm == bestv
                besti = jnp.where(cm > bestv, ci,
                                  jnp.where(tie, jnp.minimum(besti, ci),
                                            besti))
                bestv = jnp.maximum(bestv, cm)
                return bestv, besti
            _, besti = lax.fori_loop(
                0, _NW, scan_c,
                (jnp.full((_B, 1), _NEG, jnp.float32),
                 jnp.full((_B, 1), _M, jnp.int32)))

            def mark_c(c, carry):
                sl = pl.ds(c * _CW, _CW)
                iot = (lax.broadcasted_iota(jnp.int32, (_B, _CW), 1)
                       + c * _CW)
                mk_ref[:, sl] = jnp.where(iot == besti, 1.0, mk_ref[:, sl])
                return carry
            lax.fori_loop(0, _NW, mark_c, 0)
            return carry1
        lax.fori_loop(0, _K, tk_iter, 0)

        def p4(c, carry):
            sl = pl.ds(c * _CM, _CM)
            cur_c = cur_ref[:, :, sl]
            gate = us[:, :, None] * mk_ref[:, sl][:, None, :]
            cur_ref[:, :, sl] = cur_c + gate * (xt[:, :, None] - cur_c)
            return carry
        lax.fori_loop(0, _NC, p4, 0)
        return carry0

    lax.fori_loop(0, _S, step, 0)


def kernel(x, memory, Wq, bq, Wk, bk, Wv, bv, Wu, bu):
    xs = jnp.transpose(x, (1, 0, 2))
    memT = jnp.transpose(memory)
    outs = pl.pallas_call(
        _mem_layer_body,
        out_shape=jax.ShapeDtypeStruct((_S, _B, _D), jnp.float32),
        scratch_shapes=[pltpu.VMEM((_B, _D, _M), jnp.float32),
                        pltpu.VMEM((_B, _M), jnp.float32),
                        pltpu.VMEM((_B, _M), jnp.float32)],
        compiler_params=pltpu.CompilerParams(
            vmem_limit_bytes=62 * 1024 * 1024),
    )(xs, memT, Wq.T, Wk, Wv.T, Wu[:, :_D].T, Wu[:, _D:].T,
      bq.reshape(1, _D), bk.reshape(1, _D), bv.reshape(1, _D),
      bu.reshape(1, _D))
    return jnp.transpose(outs, (1, 0, 2))
